# baseline (speedup 1.0000x reference)
import functools
import jax
import jax.numpy as jnp
from jax import lax
from jax.experimental import pallas as pl
from jax.experimental.pallas import tpu as pltpu
from jax.experimental.pallas import tpu_sc as plsc

D = 64
SCALE = 8.0
LANES = 16
NBUF = 4
LEAD = 2
SPLIT = (104, 96)


def _make_emb_kernel(n_rows: int, row_len: int, rows_per_w: int, num_cores: int):
    n_chunks = rows_per_w * 2
    n_groups = n_chunks // NBUF
    assert n_chunks % NBUF == 0 and n_groups >= 2
    c_off = (0, SPLIT[0])
    mesh = plsc.VectorSubcoreMesh(core_axis_name="c", subcore_axis_name="s")

    @functools.partial(
        pl.kernel,
        out_type=jax.ShapeDtypeStruct((n_rows, row_len, D), jnp.float32),
        mesh=mesh,
        scratch_types=[
            pltpu.VMEM((rows_per_w, 2, SPLIT[0]), jnp.int32),
            pltpu.VMEM((NBUF, SPLIT[0], D), jnp.float32),
            pltpu.SemaphoreType.DMA((NBUF,)),
            pltpu.SemaphoreType.DMA((NBUF,)),
        ],
        compiler_params=pltpu.CompilerParams(use_tc_tiling_on_sc=False),
    )
    def _emb(x_hbm, table_hbm, out_hbm, idx_v, rows, gsem, osem):
        wid = lax.axis_index("s") * num_cores + lax.axis_index("c")
        row0 = wid * rows_per_w
        pltpu.sync_copy(
            x_hbm.at[pl.ds(row0, rows_per_w), pl.ds(0, SPLIT[0])],
            idx_v.at[:, 0],
        )
        pltpu.sync_copy(
            x_hbm.at[pl.ds(row0, rows_per_w), pl.ds(SPLIT[0], SPLIT[1])],
            idx_v.at[:, 1, pl.ds(0, SPLIT[1])],
        )

        def idx_ref(r, h):
            return idx_v.at[r, h, pl.ds(0, SPLIT[h])]

        def start_gather(r, h, b):
            sz = SPLIT[h]
            pltpu.async_copy(
                table_hbm.at[idx_ref(r, h)],
                rows.at[b, pl.ds(0, sz)],
                gsem.at[b],
            )

        def wait_gather(r, h, b):
            sz = SPLIT[h]
            pltpu.make_async_copy(
                table_hbm.at[idx_ref(r, h)],
                rows.at[b, pl.ds(0, sz)],
                gsem.at[b],
            ).wait()

        def out_copy(r, h, b):
            sz = SPLIT[h]
            return (
                rows.at[b, pl.ds(0, sz)],
                out_hbm.at[row0 + r, pl.ds(c_off[h], sz)],
                osem.at[b],
            )

        def start_out(r, h, b):
            pltpu.async_copy(*out_copy(r, h, b))

        def wait_out(r, h, b):
            pltpu.make_async_copy(*out_copy(r, h, b)).wait()

        def scale(h, b):
            def row_body(i, c):
                for q in range(D // LANES):
                    sl = pl.ds(q * LANES, LANES)
                    rows[b, i, sl] = rows[b, i, sl] * SCALE
                return c

            lax.fori_loop(0, SPLIT[h], row_body, 0, unroll=2)

        def process(r, h, b):
            wait_gather(r, h, b)
            scale(h, b)
            start_out(r, h, b)

        for j in range(LEAD):
            start_gather(j // 2, j % 2, j % NBUF)

        for b in range(NBUF):
            process(b // 2, b % 2, b)
            nj = b + LEAD
            if b >= LEAD:
                wait_out((nj - NBUF) // 2, (nj - NBUF) % 2, nj % NBUF)
            start_gather(nj // 2, nj % 2, nj % NBUF)

        def group_body(g, carry):
            for b in range(NBUF):
                j = g * NBUF + b
                r = 2 * g + b // 2
                h = b % 2
                process(r, h, b)
                nb = (b + LEAD) % NBUF
                pj = j + LEAD - NBUF
                wait_out(pj // 2, (b + LEAD) % 2, nb)
                start_gather((j + LEAD) // 2, (b + LEAD) % 2, nb)
            return carry

        lax.fori_loop(1, n_groups - 1, group_body, 0)

        g = n_groups - 1
        for b in range(NBUF):
            j = g * NBUF + b
            r = 2 * g + b // 2
            h = b % 2
            process(r, h, b)
            nj = j + LEAD
            if nj < n_chunks:
                wait_out((nj - NBUF) // 2, nj % 2, nj % NBUF)
                start_gather(nj // 2, nj % 2, nj % NBUF)

        for b in range(NBUF):
            j = g * NBUF + b
            wait_out(2 * g + b // 2, b % 2, b)

    return _emb


@jax.jit
def _kernel_impl(x, table):
    info = plsc.get_sparse_core_info()
    nw = info.num_cores * info.num_subcores
    n_rows, row_len = x.shape
    rows_per_w = n_rows // nw
    emb = _make_emb_kernel(n_rows, row_len, rows_per_w, info.num_cores)
    return emb(x.astype(jnp.int32), table)


_DEBUG_ONCE = []


def _debug_report(x, table):
    if _DEBUG_ONCE:
        return
    _DEBUG_ONCE.append(1)
    import sys
    try:
        comp = jax.jit(_kernel_impl).lower(x, table).compile()
        import re
        hlo = comp.as_text()
        for line in hlo.splitlines():
            if re.search(r"copy|ROOT|ENTRY", line):
                print("DBG HLO:", line.strip()[:200], file=sys.stderr)
    except Exception as e:
        print("DBG hlo fail:", repr(e), file=sys.stderr)


def kernel(x, table):
    _debug_report(x, table)
    return _kernel_impl(x, table)
